# baseline (speedup 1.0000x reference)
import functools

import jax
import jax.numpy as jnp
from jax import lax
from jax.experimental import pallas as pl
from jax.experimental.pallas import tpu as pltpu
from jax.experimental.pallas import tpu_sc as plsc

NC = 2
NS = 16
L = 16
NW = NC * NS
C = 64
D = 128


NB = 4
NP = 8
F1 = 0.30


def _core_split(total_chunks):
    tot16 = total_chunks // NS
    q1 = max(NP, NP * int(round(tot16 * F1 / NP)))
    q0 = tot16 - q1
    return q0, q1


def _spmm_sc(src, idx3, vals2):
    n = src.shape[0]
    total_chunks = idx3.shape[0]
    q0 = total_chunks // NS
    acc_rows = ((n + NS * C - 1) // (NS * C)) * (NS * C)
    zchunks = acc_rows // NS // C
    drain = acc_rows // NS

    mesh = plsc.VectorSubcoreMesh(core_axis_name="c", subcore_axis_name="s")

    @functools.partial(
        pl.kernel,
        out_type=jax.ShapeDtypeStruct((acc_rows, D), jnp.float32),
        mesh=mesh,
        scratch_types=[
            pltpu.VMEM_SHARED((acc_rows, D), jnp.float32),
            pltpu.VMEM((NB, C, D), jnp.float32),
            pltpu.VMEM((NP, 2, C), jnp.int32),
            pltpu.VMEM((NP, C), jnp.float32),
        ] + [pltpu.SemaphoreType.DMA] * (2 * NB + NP),
    )
    def k(src_hbm, idx_hbm, vals_hbm, out_hbm, acc, gb, ib, vb, *sems):
        gsem = sems[:NB]
        ssem = sems[NB:2 * NB]
        isem = sems[2 * NB:]
        c = lax.axis_index("c")
        s = lax.axis_index("s")

        @pl.when(c == 0)
        def _tile_body():
            _body(s, src_hbm, idx_hbm, vals_hbm, out_hbm, acc, gb, ib, vb,
                  gsem, ssem, isem)

    def _body(s, src_hbm, idx_hbm, vals_hbm, out_hbm, acc, gb, ib, vb,
              gsem, ssem, isem):
        zero16 = jnp.zeros((L,), jnp.float32)

        def zrow(i, carry):
            for j in range(D // L):
                gb[0, i, pl.ds(j * L, L)] = zero16
            return carry

        with jax.named_scope("zero_phase"):
            lax.fori_loop(0, C, zrow, 0)
            zbase = s * (acc_rows // NS)
            for z in range(zchunks):
                pltpu.sync_copy(gb.at[0], acc.at[pl.ds(zbase + z * C, C)])
            plsc.subcore_barrier()

        base_chunk = s * q0
        quads = q0 // NP

        def istart(i, m):
            g = base_chunk + i
            pltpu.async_copy(idx_hbm.at[g], ib.at[m], isem[m])
            pltpu.async_copy(vals_hbm.at[g], vb.at[m], isem[m])

        def iwait(i, m):
            g = base_chunk + i
            pltpu.make_async_copy(idx_hbm.at[g], ib.at[m], isem[m]).wait()
            pltpu.make_async_copy(vals_hbm.at[g], vb.at[m], isem[m]).wait()

        def gstart(m, kk):
            pltpu.async_copy(src_hbm.at[ib.at[m, 0]], gb.at[kk], gsem[kk])

        def gwait(m, kk):
            pltpu.make_async_copy(src_hbm.at[ib.at[m, 0]], gb.at[kk],
                                  gsem[kk]).wait()

        def sstart(m, kk):
            pltpu.async_copy(gb.at[kk], acc.at[ib.at[m, 1]], ssem[kk],
                             add=True)

        def swait(m, kk):
            pltpu.make_async_copy(gb.at[kk], acc.at[ib.at[m, 1]],
                                  ssem[kk]).wait()

        def scale(m, kk):
            def sgroup(g, carry2):
                v16 = vb[m, pl.ds(g * L, L)]
                for rloc in range(L):
                    sc = v16[rloc]
                    r = g * L + rloc
                    for j in range(D // L):
                        sl = pl.ds(j * L, L)
                        gb[kk, r, sl] = gb[kk, r, sl] * sc
                return carry2

            lax.fori_loop(0, C // L, sgroup, 0)

        for m in range(NP):
            istart(m, m)
        for k2 in range(NB):
            iwait(k2, k2)
            gstart(k2, k2)

        def group(t, carry):
            i0 = t * NP
            more = t < quads - 1

            for j in range(NB):
                gwait(j, j); scale(j, j); sstart(j, j)
            for k2 in range(NB):
                swait(k2, k2)

                @pl.when(more)
                def _(k2=k2):
                    istart(i0 + NP + k2, k2)

                iwait(i0 + NB + k2, NB + k2)
                gstart(NB + k2, k2)

            for j in range(NB):
                gwait(NB + j, j); scale(NB + j, j); sstart(NB + j, j)
            for k2 in range(NB):
                swait(NB + k2, k2)

                @pl.when(more)
                def _(k2=k2):
                    istart(i0 + NP + NB + k2, NB + k2)
                    iwait(i0 + NP + k2, k2)
                    gstart(k2, k2)

            return carry

        with jax.named_scope("edge_loop"):
            lax.fori_loop(0, quads, group, 0)
        with jax.named_scope("drain_phase"):
            plsc.subcore_barrier()

            dbase = s * drain

            def hstart(z):
                o = dbase + z * C
                pltpu.async_copy(gb.at[z % 2],
                                 out_hbm.at[pl.ds(o, C)],
                                 gsem[z % 2])

            def hwait(z):
                o = dbase + z * C
                pltpu.make_async_copy(gb.at[z % 2],
                                      out_hbm.at[pl.ds(o, C)],
                                      gsem[z % 2]).wait()

            for z in range(zchunks):
                if z >= 2:
                    hwait(z - 2)
                pltpu.sync_copy(acc.at[pl.ds(dbase + z * C, C)],
                                gb.at[z % 2])
                hstart(z)
            for z in range(max(0, zchunks - 2), zchunks):
                hwait(z)

    return k(src, idx3, vals2)


def _combine(partials, prev, alpha, beta, n, acc_rows):
    bn = 2048
    nb = (n + bn - 1) // bn

    def body(a_ref, p_ref, o_ref):
        o_ref[...] = alpha * a_ref[...] + beta * p_ref[...]

    return pl.pallas_call(
        body,
        grid=(nb,),
        in_specs=[
            pl.BlockSpec((bn, D), lambda i: (i, 0)),
            pl.BlockSpec((bn, D), lambda i: (i, 0)),
        ],
        out_specs=pl.BlockSpec((bn, D), lambda i: (i, 0)),
        out_shape=jax.ShapeDtypeStruct((n, D), jnp.float32),
    )(partials, prev)


def _final(xi, t1, t2, p3, w, b, n, acc_rows):
    bn = 2048
    nb = (n + bn - 1) // bn

    def body(x_ref, t1_ref, t2_ref, pa_ref, w_ref, b_ref, o_ref):
        t1b = t1_ref[...]
        acc = jnp.dot(x_ref[...], w_ref[0], preferred_element_type=jnp.float32)
        acc += jnp.dot(t1b, w_ref[1], preferred_element_type=jnp.float32)
        acc += jnp.dot(t2_ref[...], w_ref[2], preferred_element_type=jnp.float32)
        t3b = 2.0 * pa_ref[...] - t1b
        acc += jnp.dot(t3b, w_ref[3], preferred_element_type=jnp.float32)
        o_ref[...] = jnp.maximum(acc + b_ref[...], 0.0)

    return pl.pallas_call(
        body,
        grid=(nb,),
        in_specs=[
            pl.BlockSpec((bn, D), lambda i: (i, 0)),
            pl.BlockSpec((bn, D), lambda i: (i, 0)),
            pl.BlockSpec((bn, D), lambda i: (i, 0)),
            pl.BlockSpec((bn, D), lambda i: (i, 0)),
            pl.BlockSpec((4, D, D), lambda i: (0, 0, 0)),
            pl.BlockSpec((1, D), lambda i: (0, 0)),
        ],
        out_specs=pl.BlockSpec((bn, D), lambda i: (i, 0)),
        out_shape=jax.ShapeDtypeStruct((n, D), jnp.float32),
    )(xi, t1, t2, p3, w, b)


def kernel(x, lap_indices, lap_values, W, b):
    n = x.shape[1]
    e = lap_indices.shape[1]
    rows = lap_indices[0].astype(jnp.int32)
    cols = lap_indices[1].astype(jnp.int32)
    vals = lap_values.astype(jnp.float32)
    rnd = NS * NP * C
    ep = ((e + rnd - 1) // rnd) * rnd
    pad = ep - e
    if pad:
        rows = jnp.pad(rows, (0, pad))
        cols = jnp.pad(cols, (0, pad))
        vals = jnp.pad(vals, (0, pad))
    total_chunks = ep // C
    idx3 = jnp.stack([cols.reshape(total_chunks, C),
                      rows.reshape(total_chunks, C)], axis=1)
    vals2 = vals.reshape(total_chunks, C)
    b2 = b.reshape(1, D).astype(jnp.float32)
    w = W.astype(jnp.float32)

    acc_rows = ((n + NS * C - 1) // (NS * C)) * (NS * C)
    outs = []
    for i in range(x.shape[0]):
        xi = x[i]
        p1 = _spmm_sc(xi, idx3, vals2)
        t1 = _combine(p1, xi, 1.0, 0.0, n, acc_rows)
        p2 = _spmm_sc(t1, idx3, vals2)
        t2 = _combine(p2, xi, 2.0, -1.0, n, acc_rows)
        p3 = _spmm_sc(t2, idx3, vals2)
        outs.append(_final(xi, t1, t2, p3, w, b2, n, acc_rows))
    return jnp.stack(outs, axis=0)


# speedup vs baseline: 3.7756x; 3.7756x over previous
import functools

import jax
import jax.numpy as jnp
from jax import lax
from jax.experimental import pallas as pl
from jax.experimental.pallas import tpu as pltpu
from jax.experimental.pallas import tpu_sc as plsc

NC = 2
NS = 16
L = 16
NW = NC * NS
C = 64
D = 128


NB = 4
NP = 8
F1 = 0.50


def _core_split(total_chunks):
    tot16 = total_chunks // NS
    q1 = max(NP, NP * int(round(tot16 * F1 / NP)))
    q0 = tot16 - q1
    return q0, q1


def _spmm_sc(src, idx3, vals2):
    n = src.shape[0]
    total_chunks = idx3.shape[0]
    q0, q1 = _core_split(total_chunks)
    acc_rows = ((n + NS * C - 1) // (NS * C)) * (NS * C)
    zchunks = acc_rows // NS // C
    drain = acc_rows // NS

    mesh = plsc.VectorSubcoreMesh(core_axis_name="c", subcore_axis_name="s")

    @functools.partial(
        pl.kernel,
        out_type=jax.ShapeDtypeStruct((NC * acc_rows, D), jnp.float32),
        mesh=mesh,
        scratch_types=[
            pltpu.VMEM_SHARED((acc_rows, D), jnp.float32),
            pltpu.VMEM((NB, C, D), jnp.float32),
            pltpu.VMEM((NP, 2, C), jnp.int32),
            pltpu.VMEM((NP, C), jnp.float32),
        ] + [pltpu.SemaphoreType.DMA] * (2 * NB + NP),
    )
    def k(src_hbm, idx_hbm, vals_hbm, out_hbm, acc, gb, ib, vb, *sems):
        gsem = sems[:NB]
        ssem = sems[NB:2 * NB]
        isem = sems[2 * NB:]
        c = lax.axis_index("c")
        s = lax.axis_index("s")

        zero16 = jnp.zeros((L,), jnp.float32)

        def zrow(i, carry):
            for j in range(D // L):
                gb[0, i, pl.ds(j * L, L)] = zero16
            return carry

        with jax.named_scope("zero_phase"):
            lax.fori_loop(0, C, zrow, 0)
            zbase = s * (acc_rows // NS)
            for z in range(zchunks):
                pltpu.sync_copy(gb.at[0], acc.at[pl.ds(zbase + z * C, C)])
            plsc.subcore_barrier()

        base_chunk = jnp.where(c == 0, s * q0, NS * q0 + s * q1)
        nloc = jnp.where(c == 0, q0, q1)
        quads = nloc // NP

        def istart(i, m):
            g = base_chunk + i
            pltpu.async_copy(idx_hbm.at[g], ib.at[m], isem[m])
            pltpu.async_copy(vals_hbm.at[g], vb.at[m], isem[m])

        def iwait(i, m):
            g = base_chunk + i
            pltpu.make_async_copy(idx_hbm.at[g], ib.at[m], isem[m]).wait()
            pltpu.make_async_copy(vals_hbm.at[g], vb.at[m], isem[m]).wait()

        def gstart(m, kk):
            pltpu.async_copy(src_hbm.at[ib.at[m, 0]], gb.at[kk], gsem[kk])

        def gwait(m, kk):
            pltpu.make_async_copy(src_hbm.at[ib.at[m, 0]], gb.at[kk],
                                  gsem[kk]).wait()

        def sstart(m, kk):
            pltpu.async_copy(gb.at[kk], acc.at[ib.at[m, 1]], ssem[kk],
                             add=True)

        def swait(m, kk):
            pltpu.make_async_copy(gb.at[kk], acc.at[ib.at[m, 1]],
                                  ssem[kk]).wait()

        def scale(m, kk):
            def sgroup(g, carry2):
                v16 = vb[m, pl.ds(g * L, L)]
                for rloc in range(L):
                    sc = v16[rloc]
                    r = g * L + rloc
                    for j in range(D // L):
                        sl = pl.ds(j * L, L)
                        gb[kk, r, sl] = gb[kk, r, sl] * sc
                return carry2

            lax.fori_loop(0, C // L, sgroup, 0)

        for m in range(NP):
            istart(m, m)
        for k2 in range(NB):
            iwait(k2, k2)
            gstart(k2, k2)

        def group(t, carry):
            i0 = t * NP
            more = t < quads - 1

            for j in range(NB):
                gwait(j, j); scale(j, j); sstart(j, j)
            for k2 in range(NB):
                swait(k2, k2)

                @pl.when(more)
                def _(k2=k2):
                    istart(i0 + NP + k2, k2)

                iwait(i0 + NB + k2, NB + k2)
                gstart(NB + k2, k2)

            for j in range(NB):
                gwait(NB + j, j); scale(NB + j, j); sstart(NB + j, j)
            for k2 in range(NB):
                swait(NB + k2, k2)

                @pl.when(more)
                def _(k2=k2):
                    istart(i0 + NP + NB + k2, NB + k2)
                    iwait(i0 + NP + k2, k2)
                    gstart(k2, k2)

            return carry

        with jax.named_scope("edge_loop"):
            lax.fori_loop(0, quads, group, 0)
        with jax.named_scope("drain_phase"):
            plsc.subcore_barrier()

            dbase = s * drain

            def hstart(z):
                o = dbase + z * C
                pltpu.async_copy(gb.at[z % 2],
                                 out_hbm.at[pl.ds(c * acc_rows + o, C)],
                                 gsem[z % 2])

            def hwait(z):
                o = dbase + z * C
                pltpu.make_async_copy(gb.at[z % 2],
                                      out_hbm.at[pl.ds(c * acc_rows + o, C)],
                                      gsem[z % 2]).wait()

            for z in range(zchunks):
                if z >= 2:
                    hwait(z - 2)
                pltpu.sync_copy(acc.at[pl.ds(dbase + z * C, C)],
                                gb.at[z % 2])
                hstart(z)
            for z in range(max(0, zchunks - 2), zchunks):
                hwait(z)

    return k(src, idx3, vals2)


def _combine(partials, prev, alpha, beta, n, acc_rows):
    bn = 2048
    nb = (n + bn - 1) // bn
    off = acc_rows // bn

    def body(a_ref, b_ref, p_ref, o_ref):
        o_ref[...] = (alpha * (a_ref[...] + b_ref[...])
                      + beta * p_ref[...])

    return pl.pallas_call(
        body,
        grid=(nb,),
        in_specs=[
            pl.BlockSpec((bn, D), lambda i: (i, 0)),
            pl.BlockSpec((bn, D), lambda i: (i + off, 0)),
            pl.BlockSpec((bn, D), lambda i: (i, 0)),
        ],
        out_specs=pl.BlockSpec((bn, D), lambda i: (i, 0)),
        out_shape=jax.ShapeDtypeStruct((n, D), jnp.float32),
    )(partials, partials, prev)


def _final(xi, t1, t2, p3, w, b, n, acc_rows):
    bn = 2048
    nb = (n + bn - 1) // bn
    off = acc_rows // bn

    def body(x_ref, t1_ref, t2_ref, pa_ref, pb_ref, w_ref, b_ref, o_ref):
        t1b = t1_ref[...]
        acc = jnp.dot(x_ref[...], w_ref[0], preferred_element_type=jnp.float32)
        acc += jnp.dot(t1b, w_ref[1], preferred_element_type=jnp.float32)
        acc += jnp.dot(t2_ref[...], w_ref[2], preferred_element_type=jnp.float32)
        t3b = 2.0 * (pa_ref[...] + pb_ref[...]) - t1b
        acc += jnp.dot(t3b, w_ref[3], preferred_element_type=jnp.float32)
        o_ref[...] = jnp.maximum(acc + b_ref[...], 0.0)

    return pl.pallas_call(
        body,
        grid=(nb,),
        in_specs=[
            pl.BlockSpec((bn, D), lambda i: (i, 0)),
            pl.BlockSpec((bn, D), lambda i: (i, 0)),
            pl.BlockSpec((bn, D), lambda i: (i, 0)),
            pl.BlockSpec((bn, D), lambda i: (i, 0)),
            pl.BlockSpec((bn, D), lambda i: (i + off, 0)),
            pl.BlockSpec((4, D, D), lambda i: (0, 0, 0)),
            pl.BlockSpec((1, D), lambda i: (0, 0)),
        ],
        out_specs=pl.BlockSpec((bn, D), lambda i: (i, 0)),
        out_shape=jax.ShapeDtypeStruct((n, D), jnp.float32),
    )(xi, t1, t2, p3, p3, w, b)


def kernel(x, lap_indices, lap_values, W, b):
    n = x.shape[1]
    e = lap_indices.shape[1]
    rows = lap_indices[0].astype(jnp.int32)
    cols = lap_indices[1].astype(jnp.int32)
    vals = lap_values.astype(jnp.float32)
    acc_rows = ((n + NS * C - 1) // (NS * C)) * (NS * C)
    rnd = NS * NP * C
    ep = ((e + rnd - 1) // rnd) * rnd
    pad = ep - e
    if pad:
        ar = jnp.arange(pad, dtype=jnp.int32)
        rows = jnp.concatenate([rows, ar % acc_rows])
        cols = jnp.concatenate([cols, ar % n])
        vals = jnp.pad(vals, (0, pad))
    total_chunks = ep // C
    idx3 = jnp.stack([cols.reshape(total_chunks, C),
                      rows.reshape(total_chunks, C)], axis=1)
    vals2 = vals.reshape(total_chunks, C)
    b2 = b.reshape(1, D).astype(jnp.float32)
    w = W.astype(jnp.float32)

    outs = []
    for i in range(x.shape[0]):
        xi = x[i]
        p1 = _spmm_sc(xi, idx3, vals2)
        t1 = _combine(p1, xi, 1.0, 0.0, n, acc_rows)
        p2 = _spmm_sc(t1, idx3, vals2)
        t2 = _combine(p2, xi, 2.0, -1.0, n, acc_rows)
        p3 = _spmm_sc(t2, idx3, vals2)
        outs.append(_final(xi, t1, t2, p3, w, b2, n, acc_rows))
    return jnp.stack(outs, axis=0)


# speedup vs baseline: 4.1901x; 1.1098x over previous
import functools

import jax
import jax.numpy as jnp
from jax import lax
from jax.experimental import pallas as pl
from jax.experimental.pallas import tpu as pltpu
from jax.experimental.pallas import tpu_sc as plsc

NC = 2
NS = 16
L = 16
NW = NC * NS
C = 128
D = 128


NB = 2
NP = 4
F1 = 0.50


def _core_split(total_chunks):
    tot16 = total_chunks // NS
    q1 = max(NP, NP * int(round(tot16 * F1 / NP)))
    q0 = tot16 - q1
    return q0, q1


def _spmm_sc(src, idx3, vals2):
    n = src.shape[0]
    total_chunks = idx3.shape[0]
    q0, q1 = _core_split(total_chunks)
    acc_rows = ((n + NS * C - 1) // (NS * C)) * (NS * C)
    zchunks = acc_rows // NS // C
    drain = acc_rows // NS

    mesh = plsc.VectorSubcoreMesh(core_axis_name="c", subcore_axis_name="s")

    @functools.partial(
        pl.kernel,
        out_type=jax.ShapeDtypeStruct((NC * acc_rows, D), jnp.float32),
        mesh=mesh,
        scratch_types=[
            pltpu.VMEM_SHARED((acc_rows, D), jnp.float32),
            pltpu.VMEM((NB, C, D), jnp.float32),
            pltpu.VMEM((NP, 2, C), jnp.int32),
            pltpu.VMEM((NP, C), jnp.float32),
        ] + [pltpu.SemaphoreType.DMA] * (2 * NB + NP),
    )
    def k(src_hbm, idx_hbm, vals_hbm, out_hbm, acc, gb, ib, vb, *sems):
        gsem = sems[:NB]
        ssem = sems[NB:2 * NB]
        isem = sems[2 * NB:]
        c = lax.axis_index("c")
        s = lax.axis_index("s")

        zero16 = jnp.zeros((L,), jnp.float32)

        def zrow(i, carry):
            for j in range(D // L):
                gb[0, i, pl.ds(j * L, L)] = zero16
            return carry

        with jax.named_scope("zero_phase"):
            lax.fori_loop(0, C, zrow, 0)
            zbase = s * (acc_rows // NS)
            for z in range(zchunks):
                pltpu.sync_copy(gb.at[0], acc.at[pl.ds(zbase + z * C, C)])
            plsc.subcore_barrier()

        base_chunk = jnp.where(c == 0, s * q0, NS * q0 + s * q1)
        nloc = jnp.where(c == 0, q0, q1)
        quads = nloc // NP

        def istart(i, m):
            g = base_chunk + i
            pltpu.async_copy(idx_hbm.at[g], ib.at[m], isem[m])
            pltpu.async_copy(vals_hbm.at[g], vb.at[m], isem[m])

        def iwait(i, m):
            g = base_chunk + i
            pltpu.make_async_copy(idx_hbm.at[g], ib.at[m], isem[m]).wait()
            pltpu.make_async_copy(vals_hbm.at[g], vb.at[m], isem[m]).wait()

        def gstart(m, kk):
            pltpu.async_copy(src_hbm.at[ib.at[m, 0]], gb.at[kk], gsem[kk])

        def gwait(m, kk):
            pltpu.make_async_copy(src_hbm.at[ib.at[m, 0]], gb.at[kk],
                                  gsem[kk]).wait()

        def sstart(m, kk):
            pltpu.async_copy(gb.at[kk], acc.at[ib.at[m, 1]], ssem[kk],
                             add=True)

        def swait(m, kk):
            pltpu.make_async_copy(gb.at[kk], acc.at[ib.at[m, 1]],
                                  ssem[kk]).wait()

        def scale(m, kk):
            def sgroup(g, carry2):
                v16 = vb[m, pl.ds(g * L, L)]
                for rloc in range(L):
                    sc = v16[rloc]
                    r = g * L + rloc
                    for j in range(D // L):
                        sl = pl.ds(j * L, L)
                        gb[kk, r, sl] = gb[kk, r, sl] * sc
                return carry2

            lax.fori_loop(0, C // L, sgroup, 0)

        for m in range(NP):
            istart(m, m)
        for k2 in range(NB):
            iwait(k2, k2)
            gstart(k2, k2)

        def group(t, carry):
            i0 = t * NP
            more = t < quads - 1

            for j in range(NB):
                gwait(j, j); scale(j, j); sstart(j, j)
            for k2 in range(NB):
                swait(k2, k2)

                @pl.when(more)
                def _(k2=k2):
                    istart(i0 + NP + k2, k2)

                iwait(i0 + NB + k2, NB + k2)
                gstart(NB + k2, k2)

            for j in range(NB):
                gwait(NB + j, j); scale(NB + j, j); sstart(NB + j, j)
            for k2 in range(NB):
                swait(NB + k2, k2)

                @pl.when(more)
                def _(k2=k2):
                    istart(i0 + NP + NB + k2, NB + k2)
                    iwait(i0 + NP + k2, k2)
                    gstart(k2, k2)

            return carry

        with jax.named_scope("edge_loop"):
            lax.fori_loop(0, quads, group, 0)
        with jax.named_scope("drain_phase"):
            plsc.subcore_barrier()

            dbase = s * drain

            def hstart(z):
                o = dbase + z * C
                pltpu.async_copy(gb.at[z % 2],
                                 out_hbm.at[pl.ds(c * acc_rows + o, C)],
                                 gsem[z % 2])

            def hwait(z):
                o = dbase + z * C
                pltpu.make_async_copy(gb.at[z % 2],
                                      out_hbm.at[pl.ds(c * acc_rows + o, C)],
                                      gsem[z % 2]).wait()

            for z in range(zchunks):
                if z >= 2:
                    hwait(z - 2)
                pltpu.sync_copy(acc.at[pl.ds(dbase + z * C, C)],
                                gb.at[z % 2])
                hstart(z)
            for z in range(max(0, zchunks - 2), zchunks):
                hwait(z)

    return k(src, idx3, vals2)


def _combine(partials, prev, alpha, beta, n, acc_rows):
    bn = 2048
    nb = (n + bn - 1) // bn
    off = acc_rows // bn

    def body(a_ref, b_ref, p_ref, o_ref):
        o_ref[...] = (alpha * (a_ref[...] + b_ref[...])
                      + beta * p_ref[...])

    return pl.pallas_call(
        body,
        grid=(nb,),
        in_specs=[
            pl.BlockSpec((bn, D), lambda i: (i, 0)),
            pl.BlockSpec((bn, D), lambda i: (i + off, 0)),
            pl.BlockSpec((bn, D), lambda i: (i, 0)),
        ],
        out_specs=pl.BlockSpec((bn, D), lambda i: (i, 0)),
        out_shape=jax.ShapeDtypeStruct((n, D), jnp.float32),
    )(partials, partials, prev)


def _final(xi, t1, t2, p3, w, b, n, acc_rows):
    bn = 2048
    nb = (n + bn - 1) // bn
    off = acc_rows // bn

    def body(x_ref, t1_ref, t2_ref, pa_ref, pb_ref, w_ref, b_ref, o_ref):
        t1b = t1_ref[...]
        acc = jnp.dot(x_ref[...], w_ref[0], preferred_element_type=jnp.float32)
        acc += jnp.dot(t1b, w_ref[1], preferred_element_type=jnp.float32)
        acc += jnp.dot(t2_ref[...], w_ref[2], preferred_element_type=jnp.float32)
        t3b = 2.0 * (pa_ref[...] + pb_ref[...]) - t1b
        acc += jnp.dot(t3b, w_ref[3], preferred_element_type=jnp.float32)
        o_ref[...] = jnp.maximum(acc + b_ref[...], 0.0)

    return pl.pallas_call(
        body,
        grid=(nb,),
        in_specs=[
            pl.BlockSpec((bn, D), lambda i: (i, 0)),
            pl.BlockSpec((bn, D), lambda i: (i, 0)),
            pl.BlockSpec((bn, D), lambda i: (i, 0)),
            pl.BlockSpec((bn, D), lambda i: (i, 0)),
            pl.BlockSpec((bn, D), lambda i: (i + off, 0)),
            pl.BlockSpec((4, D, D), lambda i: (0, 0, 0)),
            pl.BlockSpec((1, D), lambda i: (0, 0)),
        ],
        out_specs=pl.BlockSpec((bn, D), lambda i: (i, 0)),
        out_shape=jax.ShapeDtypeStruct((n, D), jnp.float32),
    )(xi, t1, t2, p3, p3, w, b)


def kernel(x, lap_indices, lap_values, W, b):
    n = x.shape[1]
    e = lap_indices.shape[1]
    rows = lap_indices[0].astype(jnp.int32)
    cols = lap_indices[1].astype(jnp.int32)
    vals = lap_values.astype(jnp.float32)
    acc_rows = ((n + NS * C - 1) // (NS * C)) * (NS * C)
    rnd = NS * NP * C
    ep = ((e + rnd - 1) // rnd) * rnd
    pad = ep - e
    if pad:
        ar = jnp.arange(pad, dtype=jnp.int32)
        rows = jnp.concatenate([rows, ar % acc_rows])
        cols = jnp.concatenate([cols, ar % n])
        vals = jnp.pad(vals, (0, pad))
    total_chunks = ep // C
    idx3 = jnp.stack([cols.reshape(total_chunks, C),
                      rows.reshape(total_chunks, C)], axis=1)
    vals2 = vals.reshape(total_chunks, C)
    b2 = b.reshape(1, D).astype(jnp.float32)
    w = W.astype(jnp.float32)

    outs = []
    for i in range(x.shape[0]):
        xi = x[i]
        p1 = _spmm_sc(xi, idx3, vals2)
        t1 = _combine(p1, xi, 1.0, 0.0, n, acc_rows)
        p2 = _spmm_sc(t1, idx3, vals2)
        t2 = _combine(p2, xi, 2.0, -1.0, n, acc_rows)
        p3 = _spmm_sc(t2, idx3, vals2)
        outs.append(_final(xi, t1, t2, p3, w, b2, n, acc_rows))
    return jnp.stack(outs, axis=0)


# speedup vs baseline: 4.1967x; 1.0016x over previous
import functools

import jax
import jax.numpy as jnp
from jax import lax
from jax.experimental import pallas as pl
from jax.experimental.pallas import tpu as pltpu
from jax.experimental.pallas import tpu_sc as plsc

NC = 2
NS = 16
L = 16
NW = NC * NS
C = 128
D = 128


NB = 2
NP = 4
F1 = 0.50


def _core_split(total_chunks):
    tot16 = total_chunks // NS
    q1 = max(NP, NP * int(round(tot16 * F1 / NP)))
    q0 = tot16 - q1
    return q0, q1


def _spmm_sc(src, cols2, rows2, vals2):
    n = src.shape[0]
    total_chunks = cols2.shape[0]
    q0, q1 = _core_split(total_chunks)
    acc_rows = ((n + NS * C - 1) // (NS * C)) * (NS * C)
    zchunks = acc_rows // NS // C
    drain = acc_rows // NS

    mesh = plsc.VectorSubcoreMesh(core_axis_name="c", subcore_axis_name="s")

    @functools.partial(
        pl.kernel,
        out_type=jax.ShapeDtypeStruct((NC * acc_rows, D), jnp.float32),
        mesh=mesh,
        scratch_types=[
            pltpu.VMEM_SHARED((acc_rows, D), jnp.float32),
            pltpu.VMEM((NB, C, D), jnp.float32),
            pltpu.VMEM((NP, C), jnp.int32),
            pltpu.VMEM((NP, C), jnp.int32),
            pltpu.VMEM((NP, C), jnp.float32),
        ] + [pltpu.SemaphoreType.DMA] * (2 * NB + NP),
    )
    def k(src_hbm, cols_hbm, rows_hbm, vals_hbm, out_hbm,
          acc, gb, cb, rb, vb, *sems):
        gsem = sems[:NB]
        ssem = sems[NB:2 * NB]
        isem = sems[2 * NB:]
        c = lax.axis_index("c")
        s = lax.axis_index("s")

        zero16 = jnp.zeros((L,), jnp.float32)

        def zrow(i, carry):
            for j in range(D // L):
                gb[0, i, pl.ds(j * L, L)] = zero16
            return carry

        with jax.named_scope("zero_phase"):
            lax.fori_loop(0, C, zrow, 0)
            zbase = s * (acc_rows // NS)
            for z in range(zchunks):
                pltpu.sync_copy(gb.at[0], acc.at[pl.ds(zbase + z * C, C)])
            plsc.subcore_barrier()

        base_chunk = jnp.where(c == 0, s * q0, NS * q0 + s * q1)
        nloc = jnp.where(c == 0, q0, q1)
        quads = nloc // NP

        def istart(i, m):
            g = base_chunk + i
            pltpu.async_copy(cols_hbm.at[g], cb.at[m], isem[m])
            pltpu.async_copy(rows_hbm.at[g], rb.at[m], isem[m])
            pltpu.async_copy(vals_hbm.at[g], vb.at[m], isem[m])

        def iwait(i, m):
            g = base_chunk + i
            pltpu.make_async_copy(cols_hbm.at[g], cb.at[m], isem[m]).wait()
            pltpu.make_async_copy(rows_hbm.at[g], rb.at[m], isem[m]).wait()
            pltpu.make_async_copy(vals_hbm.at[g], vb.at[m], isem[m]).wait()

        def gstart(m, kk):
            pltpu.async_copy(src_hbm.at[cb.at[m]], gb.at[kk], gsem[kk])

        def gwait(m, kk):
            pltpu.make_async_copy(src_hbm.at[cb.at[m]], gb.at[kk],
                                  gsem[kk]).wait()

        def sstart(m, kk):
            pltpu.async_copy(gb.at[kk], acc.at[rb.at[m]], ssem[kk],
                             add=True)

        def swait(m, kk):
            pltpu.make_async_copy(gb.at[kk], acc.at[rb.at[m]],
                                  ssem[kk]).wait()

        def scale(m, kk):
            def sgroup(g, carry2):
                v16 = vb[m, pl.ds(g * L, L)]
                for rloc in range(L):
                    sc = v16[rloc]
                    r = g * L + rloc
                    for j in range(D // L):
                        sl = pl.ds(j * L, L)
                        gb[kk, r, sl] = gb[kk, r, sl] * sc
                return carry2

            lax.fori_loop(0, C // L, sgroup, 0)

        for m in range(NP):
            istart(m, m)
        for k2 in range(NB):
            iwait(k2, k2)
            gstart(k2, k2)

        def group(t, carry):
            i0 = t * NP
            more = t < quads - 1

            for j in range(NB):
                gwait(j, j); scale(j, j); sstart(j, j)
            for k2 in range(NB):
                swait(k2, k2)

                @pl.when(more)
                def _(k2=k2):
                    istart(i0 + NP + k2, k2)

                iwait(i0 + NB + k2, NB + k2)
                gstart(NB + k2, k2)

            for j in range(NB):
                gwait(NB + j, j); scale(NB + j, j); sstart(NB + j, j)
            for k2 in range(NB):
                swait(NB + k2, k2)

                @pl.when(more)
                def _(k2=k2):
                    istart(i0 + NP + NB + k2, NB + k2)
                    iwait(i0 + NP + k2, k2)
                    gstart(k2, k2)

            return carry

        with jax.named_scope("edge_loop"):
            lax.fori_loop(0, quads, group, 0)
        with jax.named_scope("drain_phase"):
            plsc.subcore_barrier()

            dbase = s * drain

            def hstart(z):
                o = dbase + z * C
                pltpu.async_copy(gb.at[z % 2],
                                 out_hbm.at[pl.ds(c * acc_rows + o, C)],
                                 gsem[z % 2])

            def hwait(z):
                o = dbase + z * C
                pltpu.make_async_copy(gb.at[z % 2],
                                      out_hbm.at[pl.ds(c * acc_rows + o, C)],
                                      gsem[z % 2]).wait()

            for z in range(zchunks):
                if z >= 2:
                    hwait(z - 2)
                pltpu.sync_copy(acc.at[pl.ds(dbase + z * C, C)],
                                gb.at[z % 2])
                hstart(z)
            for z in range(max(0, zchunks - 2), zchunks):
                hwait(z)

    return k(src, cols2, rows2, vals2)


def _combine(partials, prev, alpha, beta, n, acc_rows):
    bn = 2048
    nb = (n + bn - 1) // bn
    off = acc_rows // bn

    def body(a_ref, b_ref, p_ref, o_ref):
        o_ref[...] = (alpha * (a_ref[...] + b_ref[...])
                      + beta * p_ref[...])

    return pl.pallas_call(
        body,
        grid=(nb,),
        in_specs=[
            pl.BlockSpec((bn, D), lambda i: (i, 0)),
            pl.BlockSpec((bn, D), lambda i: (i + off, 0)),
            pl.BlockSpec((bn, D), lambda i: (i, 0)),
        ],
        out_specs=pl.BlockSpec((bn, D), lambda i: (i, 0)),
        out_shape=jax.ShapeDtypeStruct((n, D), jnp.float32),
    )(partials, partials, prev)


def _final(xi, t1, t2, p3, w, b, n, acc_rows):
    bn = 2048
    nb = (n + bn - 1) // bn
    off = acc_rows // bn

    def body(x_ref, t1_ref, t2_ref, pa_ref, pb_ref, w_ref, b_ref, o_ref):
        t1b = t1_ref[...]
        acc = jnp.dot(x_ref[...], w_ref[0], preferred_element_type=jnp.float32)
        acc += jnp.dot(t1b, w_ref[1], preferred_element_type=jnp.float32)
        acc += jnp.dot(t2_ref[...], w_ref[2], preferred_element_type=jnp.float32)
        t3b = 2.0 * (pa_ref[...] + pb_ref[...]) - t1b
        acc += jnp.dot(t3b, w_ref[3], preferred_element_type=jnp.float32)
        o_ref[...] = jnp.maximum(acc + b_ref[...], 0.0)

    return pl.pallas_call(
        body,
        grid=(nb,),
        in_specs=[
            pl.BlockSpec((bn, D), lambda i: (i, 0)),
            pl.BlockSpec((bn, D), lambda i: (i, 0)),
            pl.BlockSpec((bn, D), lambda i: (i, 0)),
            pl.BlockSpec((bn, D), lambda i: (i, 0)),
            pl.BlockSpec((bn, D), lambda i: (i + off, 0)),
            pl.BlockSpec((4, D, D), lambda i: (0, 0, 0)),
            pl.BlockSpec((1, D), lambda i: (0, 0)),
        ],
        out_specs=pl.BlockSpec((bn, D), lambda i: (i, 0)),
        out_shape=jax.ShapeDtypeStruct((n, D), jnp.float32),
    )(xi, t1, t2, p3, p3, w, b)


def kernel(x, lap_indices, lap_values, W, b):
    n = x.shape[1]
    e = lap_indices.shape[1]
    rows = lap_indices[0].astype(jnp.int32)
    cols = lap_indices[1].astype(jnp.int32)
    vals = lap_values.astype(jnp.float32)
    acc_rows = ((n + NS * C - 1) // (NS * C)) * (NS * C)
    rnd = NS * NP * C
    ep = ((e + rnd - 1) // rnd) * rnd
    pad = ep - e
    if pad:
        ar = jnp.arange(pad, dtype=jnp.int32)
        rows = jnp.concatenate([rows, ar % acc_rows])
        cols = jnp.concatenate([cols, ar % n])
        vals = jnp.pad(vals, (0, pad))
    total_chunks = ep // C
    cols2 = cols.reshape(total_chunks, C)
    rows2 = rows.reshape(total_chunks, C)
    vals2 = vals.reshape(total_chunks, C)
    b2 = b.reshape(1, D).astype(jnp.float32)
    w = W.astype(jnp.float32)

    outs = []
    for i in range(x.shape[0]):
        xi = x[i]
        p1 = _spmm_sc(xi, cols2, rows2, vals2)
        t1 = _combine(p1, xi, 1.0, 0.0, n, acc_rows)
        p2 = _spmm_sc(t1, cols2, rows2, vals2)
        t2 = _combine(p2, xi, 2.0, -1.0, n, acc_rows)
        p3 = _spmm_sc(t2, cols2, rows2, vals2)
        outs.append(_final(xi, t1, t2, p3, w, b2, n, acc_rows))
    return jnp.stack(outs, axis=0)


# speedup vs baseline: 4.1999x; 1.0008x over previous
import functools

import jax
import jax.numpy as jnp
from jax import lax
from jax.experimental import pallas as pl
from jax.experimental.pallas import tpu as pltpu
from jax.experimental.pallas import tpu_sc as plsc

NC = 2
NS = 16
L = 16
NW = NC * NS
C = 128
D = 128


NB = 2
NP = 4
F1 = 0.50


def _core_split(total_chunks):
    tot16 = total_chunks // NS
    q1 = max(NP, NP * int(round(tot16 * F1 / NP)))
    q0 = tot16 - q1
    return q0, q1


def _spmm_sc(src, cols2, rows2, vals2):
    n = src.shape[0]
    total_chunks = cols2.shape[0]
    q0, q1 = _core_split(total_chunks)
    acc_rows = ((n + NS * C - 1) // (NS * C)) * (NS * C)
    zchunks = acc_rows // NS // C
    drain = acc_rows // NS

    mesh = plsc.VectorSubcoreMesh(core_axis_name="c", subcore_axis_name="s")

    @functools.partial(
        pl.kernel,
        out_type=jax.ShapeDtypeStruct((NC * acc_rows, D), jnp.float32),
        mesh=mesh,
        scratch_types=[
            pltpu.VMEM_SHARED((acc_rows, D), jnp.float32),
            pltpu.VMEM((NB, C, D), jnp.float32),
            pltpu.VMEM((NP, C), jnp.int32),
            pltpu.VMEM((NP, C), jnp.int32),
            pltpu.VMEM((NP, C), jnp.float32),
        ] + [pltpu.SemaphoreType.DMA] * (2 * NB + NP),
    )
    def k(src_hbm, cols_hbm, rows_hbm, vals_hbm, out_hbm,
          acc, gb, cb, rb, vb, *sems):
        gsem = sems[:NB]
        ssem = sems[NB:2 * NB]
        isem = sems[2 * NB:]
        c = lax.axis_index("c")
        s = lax.axis_index("s")

        zero16 = jnp.zeros((L,), jnp.float32)

        def zrow(i, carry):
            for j in range(D // L):
                gb[0, i, pl.ds(j * L, L)] = zero16
            return carry

        lax.fori_loop(0, C, zrow, 0)
        zbase = s * (acc_rows // NS)
        for z in range(zchunks):
            pltpu.sync_copy(gb.at[0], acc.at[pl.ds(zbase + z * C, C)])
        plsc.subcore_barrier()

        base_chunk = jnp.where(c == 0, s * q0, NS * q0 + s * q1)
        nloc = jnp.where(c == 0, q0, q1)
        quads = nloc // NP

        def istart(i, m):
            g = base_chunk + i
            pltpu.async_copy(cols_hbm.at[g], cb.at[m], isem[m])
            pltpu.async_copy(rows_hbm.at[g], rb.at[m], isem[m])
            pltpu.async_copy(vals_hbm.at[g], vb.at[m], isem[m])

        def iwait(i, m):
            g = base_chunk + i
            pltpu.make_async_copy(cols_hbm.at[g], cb.at[m], isem[m]).wait()
            pltpu.make_async_copy(rows_hbm.at[g], rb.at[m], isem[m]).wait()
            pltpu.make_async_copy(vals_hbm.at[g], vb.at[m], isem[m]).wait()

        def gstart(m, kk):
            pltpu.async_copy(src_hbm.at[cb.at[m]], gb.at[kk], gsem[kk])

        def gwait(m, kk):
            pltpu.make_async_copy(src_hbm.at[cb.at[m]], gb.at[kk],
                                  gsem[kk]).wait()

        def sstart(m, kk):
            pltpu.async_copy(gb.at[kk], acc.at[rb.at[m]], ssem[kk],
                             add=True)

        def swait(m, kk):
            pltpu.make_async_copy(gb.at[kk], acc.at[rb.at[m]],
                                  ssem[kk]).wait()

        def scale(m, kk):
            def sgroup(g, carry2):
                v16 = vb[m, pl.ds(g * L, L)]
                for rloc in range(L):
                    sc = v16[rloc]
                    r = g * L + rloc
                    for j in range(D // L):
                        sl = pl.ds(j * L, L)
                        gb[kk, r, sl] = gb[kk, r, sl] * sc
                return carry2

            lax.fori_loop(0, C // L, sgroup, 0)

        for m in range(NP):
            istart(m, m)
        for k2 in range(NB):
            iwait(k2, k2)
            gstart(k2, k2)

        def group(t, carry):
            i0 = t * NP
            more = t < quads - 1

            for j in range(NB):
                gwait(j, j); scale(j, j); sstart(j, j)
            for k2 in range(NB):
                swait(k2, k2)

                @pl.when(more)
                def _(k2=k2):
                    istart(i0 + NP + k2, k2)

                iwait(i0 + NB + k2, NB + k2)
                gstart(NB + k2, k2)

            for j in range(NB):
                gwait(NB + j, j); scale(NB + j, j); sstart(NB + j, j)
            for k2 in range(NB):
                swait(NB + k2, k2)

                @pl.when(more)
                def _(k2=k2):
                    istart(i0 + NP + NB + k2, NB + k2)
                    iwait(i0 + NP + k2, k2)
                    gstart(k2, k2)

            return carry

        lax.fori_loop(0, quads, group, 0)
        plsc.subcore_barrier()

        dbase = s * drain

        def hstart(z):
            o = dbase + z * C
            pltpu.async_copy(gb.at[z % 2],
                             out_hbm.at[pl.ds(c * acc_rows + o, C)],
                             gsem[z % 2])

        def hwait(z):
            o = dbase + z * C
            pltpu.make_async_copy(gb.at[z % 2],
                                  out_hbm.at[pl.ds(c * acc_rows + o, C)],
                                  gsem[z % 2]).wait()

        for z in range(zchunks):
            if z >= 2:
                hwait(z - 2)
            pltpu.sync_copy(acc.at[pl.ds(dbase + z * C, C)],
                            gb.at[z % 2])
            hstart(z)
        for z in range(max(0, zchunks - 2), zchunks):
            hwait(z)

    return k(src, cols2, rows2, vals2)


def _combine(partials, prev, alpha, beta, n, acc_rows):
    bn = 2048
    nb = (n + bn - 1) // bn
    off = acc_rows // bn

    def body(a_ref, b_ref, p_ref, o_ref):
        o_ref[...] = (alpha * (a_ref[...] + b_ref[...])
                      + beta * p_ref[...])

    return pl.pallas_call(
        body,
        grid=(nb,),
        in_specs=[
            pl.BlockSpec((bn, D), lambda i: (i, 0)),
            pl.BlockSpec((bn, D), lambda i: (i + off, 0)),
            pl.BlockSpec((bn, D), lambda i: (i, 0)),
        ],
        out_specs=pl.BlockSpec((bn, D), lambda i: (i, 0)),
        out_shape=jax.ShapeDtypeStruct((n, D), jnp.float32),
    )(partials, partials, prev)


def _final(xi, t1, t2, p3, w, b, n, acc_rows):
    bn = 2048
    nb = (n + bn - 1) // bn
    off = acc_rows // bn

    def body(x_ref, t1_ref, t2_ref, pa_ref, pb_ref, w_ref, b_ref, o_ref):
        t1b = t1_ref[...]
        acc = jnp.dot(x_ref[...], w_ref[0], preferred_element_type=jnp.float32)
        acc += jnp.dot(t1b, w_ref[1], preferred_element_type=jnp.float32)
        acc += jnp.dot(t2_ref[...], w_ref[2], preferred_element_type=jnp.float32)
        t3b = 2.0 * (pa_ref[...] + pb_ref[...]) - t1b
        acc += jnp.dot(t3b, w_ref[3], preferred_element_type=jnp.float32)
        o_ref[...] = jnp.maximum(acc + b_ref[...], 0.0)

    return pl.pallas_call(
        body,
        grid=(nb,),
        in_specs=[
            pl.BlockSpec((bn, D), lambda i: (i, 0)),
            pl.BlockSpec((bn, D), lambda i: (i, 0)),
            pl.BlockSpec((bn, D), lambda i: (i, 0)),
            pl.BlockSpec((bn, D), lambda i: (i, 0)),
            pl.BlockSpec((bn, D), lambda i: (i + off, 0)),
            pl.BlockSpec((4, D, D), lambda i: (0, 0, 0)),
            pl.BlockSpec((1, D), lambda i: (0, 0)),
        ],
        out_specs=pl.BlockSpec((bn, D), lambda i: (i, 0)),
        out_shape=jax.ShapeDtypeStruct((n, D), jnp.float32),
    )(xi, t1, t2, p3, p3, w, b)


def kernel(x, lap_indices, lap_values, W, b):
    n = x.shape[1]
    e = lap_indices.shape[1]
    rows = lap_indices[0].astype(jnp.int32)
    cols = lap_indices[1].astype(jnp.int32)
    vals = lap_values.astype(jnp.float32)
    acc_rows = ((n + NS * C - 1) // (NS * C)) * (NS * C)
    rnd = NS * NP * C
    ep = ((e + rnd - 1) // rnd) * rnd
    pad = ep - e
    if pad:
        ar = jnp.arange(pad, dtype=jnp.int32)
        rows = jnp.concatenate([rows, ar % acc_rows])
        cols = jnp.concatenate([cols, ar % n])
        vals = jnp.pad(vals, (0, pad))
    total_chunks = ep // C
    cols2 = cols.reshape(total_chunks, C)
    rows2 = rows.reshape(total_chunks, C)
    vals2 = vals.reshape(total_chunks, C)
    b2 = b.reshape(1, D).astype(jnp.float32)
    w = W.astype(jnp.float32)

    outs = []
    for i in range(x.shape[0]):
        xi = x[i]
        p1 = _spmm_sc(xi, cols2, rows2, vals2)
        t1 = _combine(p1, xi, 1.0, 0.0, n, acc_rows)
        p2 = _spmm_sc(t1, cols2, rows2, vals2)
        t2 = _combine(p2, xi, 2.0, -1.0, n, acc_rows)
        p3 = _spmm_sc(t2, cols2, rows2, vals2)
        outs.append(_final(xi, t1, t2, p3, w, b2, n, acc_rows))
    return jnp.stack(outs, axis=0)
